# Initial kernel scaffold; baseline (speedup 1.0000x reference)
import jax
import jax.numpy as jnp
from jax.experimental import pallas as pl


def kernel(x, edge_index, edge_attr, We, be, W1, b1, W2, b2, ln_g, ln_b):
    raise NotImplementedError("write your pallas kernel here")


# speedup vs baseline: 2.9940x; 2.9940x over previous
import functools

import jax
import jax.numpy as jnp
from jax import lax
from jax.experimental import pallas as pl
from jax.experimental.pallas import tpu as pltpu
from jax.experimental.pallas import tpu_sc as plsc

_NC = 2
_NS = 16
_K = 80


def _ea_body(ea_ref, we_ref, be_ref, out_ref):
    out_ref[...] = (
        jnp.dot(ea_ref[...], we_ref[...], preferred_element_type=jnp.float32)
        + be_ref[...]
    )


def _edge_proj(edge_attr, we, be):
    e, de = edge_attr.shape
    d = we.shape[1]
    be_block = 4000
    return pl.pallas_call(
        _ea_body,
        grid=(e // be_block,),
        in_specs=[
            pl.BlockSpec((be_block, de), lambda i: (i, 0)),
            pl.BlockSpec((de, d), lambda i: (0, 0)),
            pl.BlockSpec((1, d), lambda i: (0, 0)),
        ],
        out_specs=pl.BlockSpec((be_block, d), lambda i: (i, 0)),
        out_shape=jax.ShapeDtypeStruct((e, d), jnp.float32),
    )(edge_attr, we, be.reshape(1, d))


def _mlp_body(h_ref, a0_ref, a1_ref, w1_ref, b1_ref, w2_ref, b2_ref, g_ref,
              bb_ref, out_ref):
    h = h_ref[...]
    z = h + a0_ref[...] + a1_ref[...]
    t = jnp.maximum(
        jnp.dot(z, w1_ref[...], preferred_element_type=jnp.float32) + b1_ref[...],
        0.0,
    )
    o = jnp.dot(t, w2_ref[...], preferred_element_type=jnp.float32) + b2_ref[...]
    m = jnp.mean(o, axis=-1, keepdims=True)
    v = jnp.mean((o - m) ** 2, axis=-1, keepdims=True)
    o = (o - m) / jnp.sqrt(v + 1e-5) * g_ref[...] + bb_ref[...]
    out_ref[...] = jnp.maximum(o + h, 0.0)


def _mlp(h, a0, a1, w1, b1, w2, b2, g, bb):
    n, d = h.shape
    dh = w1.shape[1]
    bn = 1000
    full = lambda i: (0, 0)
    row = lambda i: (i, 0)
    return pl.pallas_call(
        _mlp_body,
        grid=(n // bn,),
        in_specs=[
            pl.BlockSpec((bn, d), row),
            pl.BlockSpec((bn, d), row),
            pl.BlockSpec((bn, d), row),
            pl.BlockSpec((d, dh), full),
            pl.BlockSpec((1, dh), full),
            pl.BlockSpec((dh, d), full),
            pl.BlockSpec((1, d), full),
            pl.BlockSpec((1, d), full),
            pl.BlockSpec((1, d), full),
        ],
        out_specs=pl.BlockSpec((bn, d), row),
        out_shape=jax.ShapeDtypeStruct((n, d), jnp.float32),
    )(h, a0, a1, w1, b1.reshape(1, dh), w2, b2.reshape(1, d), g.reshape(1, d),
      bb.reshape(1, d))


def _make_sc_agg(n, e, d):
    ept = e // (_NC * _NS)
    chunks = ept // _K
    npad = -(-n // (8 * _NS)) * (8 * _NS)
    zr = npad // _NS
    mesh = plsc.VectorSubcoreMesh(core_axis_name="c", subcore_axis_name="s")

    @functools.partial(
        pl.kernel,
        mesh=mesh,
        out_type=jax.ShapeDtypeStruct((_NC, npad, d), jnp.float32),
        scratch_types=[
            pltpu.VMEM_SHARED((npad, d), jnp.float32),
            pltpu.VMEM((_K,), jnp.int32),
            pltpu.VMEM((_K,), jnp.int32),
            pltpu.VMEM((_K, d), jnp.float32),
            pltpu.VMEM((_K, d), jnp.float32),
            pltpu.SemaphoreType.DMA,
            pltpu.SemaphoreType.DMA,
        ],
    )
    def sc_agg(h_hbm, src_hbm, dst_hbm, ea_hbm, out_hbm, agg_sp, sidx, didx,
               rows, msg, sem1, sem2):
        c = lax.axis_index("c")
        s = lax.axis_index("s")
        wid = c * _NS + s

        def zrow(k, carry):
            for j in range(d // 16):
                rows[k, pl.ds(j * 16, 16)] = jnp.zeros((16,), jnp.float32)
            return carry

        lax.fori_loop(0, _K, zrow, 0)
        zbase = s * zr
        nfull = zr // _K
        for t in range(nfull):
            pltpu.sync_copy(rows, agg_sp.at[pl.ds(zbase + t * _K, _K)])
        if zr % _K != 0:
            pltpu.sync_copy(rows, agg_sp.at[pl.ds(zbase + zr - _K, _K)])
        plsc.subcore_barrier()

        def chunk_body(i, carry):
            base = wid * ept + i * _K
            pltpu.sync_copy(src_hbm.at[pl.ds(base, _K)], sidx)
            pltpu.sync_copy(dst_hbm.at[pl.ds(base, _K)], didx)
            cp_g = pltpu.async_copy(h_hbm.at[sidx], rows, sem1)
            cp_e = pltpu.async_copy(ea_hbm.at[pl.ds(base, _K)], msg, sem2)
            cp_g.wait()
            cp_e.wait()

            def mrow(k, carry2):
                for j in range(d // 16):
                    sl = pl.ds(j * 16, 16)
                    msg[k, sl] = jnp.maximum(msg[k, sl] + rows[k, sl], 0.0)
                return carry2

            lax.fori_loop(0, _K, mrow, 0)
            pltpu.sync_copy(msg, agg_sp.at[didx], add=True)
            return carry

        lax.fori_loop(0, chunks, chunk_body, 0)
        plsc.subcore_barrier()
        pltpu.sync_copy(agg_sp.at[pl.ds(s * zr, zr)],
                        out_hbm.at[c, pl.ds(s * zr, zr)])

    return sc_agg


def kernel(x, edge_index, edge_attr, We, be, W1, b1, W2, b2, ln_g, ln_b):
    n, d = x.shape
    e = edge_index.shape[1]
    nl = We.shape[0]
    src = edge_index[0]
    dst = edge_index[1]
    sc_agg = _make_sc_agg(n, e, d)
    h = x.astype(jnp.float32)
    for i in range(nl):
        ea = _edge_proj(edge_attr, We[i], be[i])
        parts = sc_agg(h, src, dst, ea)
        h = _mlp(h, parts[0, :n], parts[1, :n], W1[i], b1[i], W2[i], b2[i],
                 ln_g[i], ln_b[i])
    return h


# speedup vs baseline: 4.0599x; 1.3560x over previous
import functools

import jax
import jax.numpy as jnp
from jax import lax
from jax.experimental import pallas as pl
from jax.experimental.pallas import tpu as pltpu
from jax.experimental.pallas import tpu_sc as plsc

_NC = 2
_NS = 16
_K = 80


def _ea_body(ea_ref, we_ref, be_ref, out_ref):
    out_ref[...] = (
        jnp.dot(ea_ref[...], we_ref[...], preferred_element_type=jnp.float32)
        + be_ref[...]
    )


def _edge_proj(edge_attr, we, be):
    e, de = edge_attr.shape
    d = we.shape[1]
    be_block = 4000
    return pl.pallas_call(
        _ea_body,
        grid=(e // be_block,),
        in_specs=[
            pl.BlockSpec((be_block, de), lambda i: (i, 0)),
            pl.BlockSpec((de, d), lambda i: (0, 0)),
            pl.BlockSpec((1, d), lambda i: (0, 0)),
        ],
        out_specs=pl.BlockSpec((be_block, d), lambda i: (i, 0)),
        out_shape=jax.ShapeDtypeStruct((e, d), jnp.float32),
    )(edge_attr, we, be.reshape(1, d))


def _mlp_body(h_ref, a0_ref, a1_ref, w1_ref, b1_ref, w2_ref, b2_ref, g_ref,
              bb_ref, out_ref):
    h = h_ref[...]
    z = h + a0_ref[...] + a1_ref[...]
    t = jnp.maximum(
        jnp.dot(z, w1_ref[...], preferred_element_type=jnp.float32) + b1_ref[...],
        0.0,
    )
    o = jnp.dot(t, w2_ref[...], preferred_element_type=jnp.float32) + b2_ref[...]
    m = jnp.mean(o, axis=-1, keepdims=True)
    v = jnp.mean((o - m) ** 2, axis=-1, keepdims=True)
    o = (o - m) / jnp.sqrt(v + 1e-5) * g_ref[...] + bb_ref[...]
    out_ref[...] = jnp.maximum(o + h, 0.0)


def _mlp(h, a0, a1, w1, b1, w2, b2, g, bb):
    n, d = h.shape
    dh = w1.shape[1]
    bn = 1000
    full = lambda i: (0, 0)
    row = lambda i: (i, 0)
    return pl.pallas_call(
        _mlp_body,
        grid=(n // bn,),
        in_specs=[
            pl.BlockSpec((bn, d), row),
            pl.BlockSpec((bn, d), row),
            pl.BlockSpec((bn, d), row),
            pl.BlockSpec((d, dh), full),
            pl.BlockSpec((1, dh), full),
            pl.BlockSpec((dh, d), full),
            pl.BlockSpec((1, d), full),
            pl.BlockSpec((1, d), full),
            pl.BlockSpec((1, d), full),
        ],
        out_specs=pl.BlockSpec((bn, d), row),
        out_shape=jax.ShapeDtypeStruct((n, d), jnp.float32),
    )(h, a0, a1, w1, b1.reshape(1, dh), w2, b2.reshape(1, d), g.reshape(1, d),
      bb.reshape(1, d))


def _make_sc_agg(n, e, d):
    ept = e // (_NC * _NS)
    chunks = ept // _K
    npad = -(-n // (8 * _NS)) * (8 * _NS)
    zr = npad // _NS
    mesh = plsc.VectorSubcoreMesh(core_axis_name="c", subcore_axis_name="s")

    vec = []
    vec += [pltpu.VMEM((_K,), jnp.int32) for _ in range(4)]
    vec += [pltpu.VMEM((_K, d), jnp.float32) for _ in range(2)]
    vec += [pltpu.VMEM((_K, d), jnp.float32) for _ in range(2)]
    vec += [pltpu.SemaphoreType.DMA for _ in range(6)]
    vec += [pltpu.VMEM_SHARED((npad, d), jnp.float32)]

    @functools.partial(
        pl.kernel,
        mesh=mesh,
        out_type=jax.ShapeDtypeStruct((_NC, npad, d), jnp.float32),
        scratch_types=vec,
    )
    def sc_agg(h_hbm, src_hbm, dst_hbm, ea_hbm, out_hbm, *bufs):
        sidx = bufs[0:2]
        didx = bufs[2:4]
        rows = bufs[4:6]
        msg = bufs[6:8]
        semg = bufs[8:10]
        seme = bufs[10:12]
        sems = bufs[12:14]
        agg_sp = bufs[14]
        c = lax.axis_index("c")
        s = lax.axis_index("s")
        wid = c * _NS + s
        ebase = wid * ept

        def zrow(k, carry):
            for j in range(d // 16):
                rows[0][k, pl.ds(j * 16, 16)] = jnp.zeros((16,), jnp.float32)
            return carry

        lax.fori_loop(0, _K, zrow, 0)
        zbase = s * zr
        for t in range(zr // _K):
            pltpu.sync_copy(rows[0], agg_sp.at[pl.ds(zbase + t * _K, _K)])
        if zr % _K != 0:
            pltpu.sync_copy(rows[0], agg_sp.at[pl.ds(zbase + zr - _K, _K)])
        plsc.subcore_barrier()

        def load_idx(i, b):
            pltpu.sync_copy(src_hbm.at[pl.ds(ebase + i * _K, _K)], sidx[b])
            pltpu.sync_copy(dst_hbm.at[pl.ds(ebase + i * _K, _K)], didx[b])

        def issue(i, b):
            g = pltpu.async_copy(h_hbm.at[sidx[b]], rows[b], semg[b])
            e = pltpu.async_copy(ea_hbm.at[pl.ds(ebase + i * _K, _K)],
                                 msg[b], seme[b])
            return g, e

        def compute(b):
            def mrow(k, carry2):
                for j in range(d // 16):
                    sl = pl.ds(j * 16, 16)
                    msg[b][k, sl] = jnp.maximum(msg[b][k, sl] + rows[b][k, sl],
                                                0.0)
                return carry2

            lax.fori_loop(0, _K, mrow, 0)

        load_idx(0, 0)
        g0, e0 = issue(0, 0)
        load_idx(1, 1)
        g1, e1 = issue(1, 1)

        def body(t, carry):
            i = 2 * t
            g0.wait()
            e0.wait()
            compute(0)
            s0 = pltpu.async_copy(msg[0], agg_sp.at[didx[0]], sems[0],
                                  add=True)
            g1.wait()
            e1.wait()
            compute(1)
            s1 = pltpu.async_copy(msg[1], agg_sp.at[didx[1]], sems[1],
                                  add=True)
            s0.wait()

            @pl.when(i + 2 < chunks)
            def _():
                load_idx(i + 2, 0)
                issue(i + 2, 0)

            s1.wait()

            @pl.when(i + 3 < chunks)
            def _():
                load_idx(i + 3, 1)
                issue(i + 3, 1)

            return carry

        lax.fori_loop(0, chunks // 2, body, 0)
        if chunks % 2 != 0:
            g0.wait()
            e0.wait()
            compute(0)
            pltpu.sync_copy(msg[0], agg_sp.at[didx[0]], add=True)
        plsc.subcore_barrier()
        pltpu.sync_copy(agg_sp.at[pl.ds(s * zr, zr)],
                        out_hbm.at[c, pl.ds(s * zr, zr)])

    return sc_agg


def kernel(x, edge_index, edge_attr, We, be, W1, b1, W2, b2, ln_g, ln_b):
    n, d = x.shape
    e = edge_index.shape[1]
    nl = We.shape[0]
    src = edge_index[0]
    dst = edge_index[1]
    sc_agg = _make_sc_agg(n, e, d)
    h = x.astype(jnp.float32)
    for i in range(nl):
        ea = _edge_proj(edge_attr, We[i], be[i])
        parts = sc_agg(h, src, dst, ea)
        h = _mlp(h, parts[0, :n], parts[1, :n], W1[i], b1[i], W2[i], b2[i],
                 ln_g[i], ln_b[i])
    return h


# speedup vs baseline: 5.1342x; 1.2646x over previous
import functools

import jax
import jax.numpy as jnp
from jax import lax
from jax.experimental import pallas as pl
from jax.experimental.pallas import tpu as pltpu
from jax.experimental.pallas import tpu_sc as plsc

_NC = 2
_NS = 16
_K = 80


def _ea_body(ea_ref, we_ref, be_ref, out_ref):
    out_ref[...] = (
        jnp.dot(ea_ref[...], we_ref[...], preferred_element_type=jnp.float32)
        + be_ref[...]
    )


def _edge_proj(edge_attr, we, be):
    e, de = edge_attr.shape
    d = we.shape[1]
    be_block = 4000
    return pl.pallas_call(
        _ea_body,
        grid=(e // be_block,),
        in_specs=[
            pl.BlockSpec((be_block, de), lambda i: (i, 0)),
            pl.BlockSpec((de, d), lambda i: (0, 0)),
            pl.BlockSpec((1, d), lambda i: (0, 0)),
        ],
        out_specs=pl.BlockSpec((be_block, d), lambda i: (i, 0)),
        out_shape=jax.ShapeDtypeStruct((e, d), jnp.float32),
    )(edge_attr, we, be.reshape(1, d))


def _mlp_body(h_ref, a0_ref, a1_ref, w1_ref, b1_ref, w2_ref, b2_ref, g_ref,
              bb_ref, out_ref):
    h = h_ref[...]
    z = h + a0_ref[...] + a1_ref[...]
    t = jnp.maximum(
        jnp.dot(z, w1_ref[...], preferred_element_type=jnp.float32) + b1_ref[...],
        0.0,
    )
    o = jnp.dot(t, w2_ref[...], preferred_element_type=jnp.float32) + b2_ref[...]
    m = jnp.mean(o, axis=-1, keepdims=True)
    v = jnp.mean((o - m) ** 2, axis=-1, keepdims=True)
    o = (o - m) / jnp.sqrt(v + 1e-5) * g_ref[...] + bb_ref[...]
    out_ref[...] = jnp.maximum(o + h, 0.0)


def _mlp(h, a0, a1, w1, b1, w2, b2, g, bb):
    n, d = h.shape
    dh = w1.shape[1]
    bn = 1000
    full = lambda i: (0, 0)
    row = lambda i: (i, 0)
    return pl.pallas_call(
        _mlp_body,
        grid=(n // bn,),
        in_specs=[
            pl.BlockSpec((bn, d), row),
            pl.BlockSpec((bn, d), row),
            pl.BlockSpec((bn, d), row),
            pl.BlockSpec((d, dh), full),
            pl.BlockSpec((1, dh), full),
            pl.BlockSpec((dh, d), full),
            pl.BlockSpec((1, d), full),
            pl.BlockSpec((1, d), full),
            pl.BlockSpec((1, d), full),
        ],
        out_specs=pl.BlockSpec((bn, d), row),
        out_shape=jax.ShapeDtypeStruct((n, d), jnp.float32),
    )(h, a0, a1, w1, b1.reshape(1, dh), w2, b2.reshape(1, d), g.reshape(1, d),
      bb.reshape(1, d))


def _make_sc_agg(n, e, d):
    ept = e // (_NC * _NS)
    chunks = ept // _K
    npad = -(-n // (8 * _NS)) * (8 * _NS)
    zr = npad // _NS
    mesh = plsc.VectorSubcoreMesh(core_axis_name="c", subcore_axis_name="s")

    vec = []
    vec += [pltpu.VMEM((_K,), jnp.int32) for _ in range(4)]
    vec += [pltpu.VMEM((_K, d), jnp.float32) for _ in range(2)]
    vec += [pltpu.VMEM((_K, d), jnp.float32) for _ in range(2)]
    vec += [pltpu.SemaphoreType.DMA for _ in range(10)]
    vec += [pltpu.VMEM_SHARED((npad, d), jnp.float32)]

    @functools.partial(
        pl.kernel,
        mesh=mesh,
        out_type=jax.ShapeDtypeStruct((_NC, npad, d), jnp.float32),
        scratch_types=vec,
    )
    def sc_agg(h_hbm, src_hbm, dst_hbm, ea_hbm, out_hbm, *bufs):
        sidx = bufs[0:2]
        didx = bufs[2:4]
        rows = bufs[4:6]
        msg = bufs[6:8]
        semg = bufs[8:10]
        seme = bufs[10:12]
        sems = bufs[12:14]
        semi = bufs[14:16]
        semid = bufs[16:18]
        agg_sp = bufs[18]
        c = lax.axis_index("c")
        s = lax.axis_index("s")
        wid = c * _NS + s
        ebase = wid * ept

        def zrow(k, carry):
            for j in range(d // 16):
                rows[0][k, pl.ds(j * 16, 16)] = jnp.zeros((16,), jnp.float32)
            return carry

        lax.fori_loop(0, _K, zrow, 0)
        zbase = s * zr
        for t in range(zr // _K):
            pltpu.sync_copy(rows[0], agg_sp.at[pl.ds(zbase + t * _K, _K)])
        if zr % _K != 0:
            pltpu.sync_copy(rows[0], agg_sp.at[pl.ds(zbase + zr - _K, _K)])
        plsc.subcore_barrier()

        def issue(i, b):
            g = pltpu.async_copy(h_hbm.at[sidx[b]], rows[b], semg[b])
            e = pltpu.async_copy(ea_hbm.at[pl.ds(ebase + i * _K, _K)],
                                 msg[b], seme[b])
            return g, e

        def compute(b):
            def mrow(k, carry2):
                for j in range(d // 16):
                    sl = pl.ds(j * 16, 16)
                    msg[b][k, sl] = jnp.maximum(msg[b][k, sl] + rows[b][k, sl],
                                                0.0)
                return carry2

            lax.fori_loop(0, _K, mrow, 0)

        def sidx_load(i, b):
            return pltpu.async_copy(src_hbm.at[pl.ds(ebase + i * _K, _K)],
                                    sidx[b], semi[b])

        def didx_load(i, b):
            return pltpu.async_copy(dst_hbm.at[pl.ds(ebase + i * _K, _K)],
                                    didx[b], semid[b])

        pltpu.sync_copy(src_hbm.at[pl.ds(ebase, _K)], sidx[0])
        didx_load(0, 0)
        g0, e0 = issue(0, 0)
        pltpu.sync_copy(src_hbm.at[pl.ds(ebase + _K, _K)], sidx[1])
        didx_load(1, 1)
        g1, e1 = issue(1, 1)

        def half(t, b):
            i = 2 * t + b
            g0.wait() if b == 0 else g1.wait()
            e0.wait() if b == 0 else e1.wait()

            @pl.when(i + 2 < chunks)
            def _():
                sidx_load(i + 2, b)

            compute(b)
            pltpu.make_async_copy(
                dst_hbm.at[pl.ds(ebase + i * _K, _K)], didx[b],
                semid[b]).wait()
            sc = pltpu.async_copy(msg[b], agg_sp.at[didx[b]], sems[b],
                                  add=True)
            sc.wait()

            @pl.when(i + 2 < chunks)
            def _():
                didx_load(i + 2, b)
                pltpu.make_async_copy(
                    src_hbm.at[pl.ds(ebase + i * _K, _K)], sidx[b],
                    semi[b]).wait()
                issue(i + 2, b)

        def body(t, carry):
            half(t, 0)
            half(t, 1)
            return carry

        lax.fori_loop(0, chunks // 2, body, 0)
        if chunks % 2 != 0:
            g0.wait()
            e0.wait()
            compute(0)
            pltpu.make_async_copy(
                dst_hbm.at[pl.ds(ebase + (chunks - 1) * _K, _K)], didx[0],
                semid[0]).wait()
            pltpu.sync_copy(msg[0], agg_sp.at[didx[0]], add=True)
        plsc.subcore_barrier()
        pltpu.sync_copy(agg_sp.at[pl.ds(s * zr, zr)],
                        out_hbm.at[c, pl.ds(s * zr, zr)])

    return sc_agg


def kernel(x, edge_index, edge_attr, We, be, W1, b1, W2, b2, ln_g, ln_b):
    n, d = x.shape
    e = edge_index.shape[1]
    nl = We.shape[0]
    src = edge_index[0]
    dst = edge_index[1]
    sc_agg = _make_sc_agg(n, e, d)
    h = x.astype(jnp.float32)
    for i in range(nl):
        ea = _edge_proj(edge_attr, We[i], be[i])
        parts = sc_agg(h, src, dst, ea)
        h = _mlp(h, parts[0, :n], parts[1, :n], W1[i], b1[i], W2[i], b2[i],
                 ln_g[i], ln_b[i])
    return h
